# baseline (speedup 1.0000x reference)
import functools

import jax
import jax.numpy as jnp
from jax import lax
from jax.experimental import pallas as pl
from jax.experimental.pallas import tpu as pltpu
from jax.experimental.pallas import tpu_sc as plsc

N_ENTITY = 1000000
N_RELATION = 1000
HIDDEN = 64
BATCH = 4096

_NC, _NS = 2, 16
_NW = _NC * _NS
_BW = BATCH // _NW

_mesh = plsc.VectorSubcoreMesh(core_axis_name="c", subcore_axis_name="s")


@functools.partial(
    pl.kernel,
    mesh=_mesh,
    compiler_params=pltpu.CompilerParams(use_tc_tiling_on_sc=True,
                                         needs_layout_passes=False),
    out_type=(
        jax.ShapeDtypeStruct((BATCH, HIDDEN), jnp.float32),
        jax.ShapeDtypeStruct((BATCH, HIDDEN), jnp.float32),
        jax.ShapeDtypeStruct((BATCH, HIDDEN), jnp.float32),
        jax.ShapeDtypeStruct((BATCH, HIDDEN), jnp.float32),
        jax.ShapeDtypeStruct((BATCH,), jnp.float32),
        jax.ShapeDtypeStruct((BATCH, 2 * HIDDEN), jnp.float32),
        jax.ShapeDtypeStruct((BATCH, HIDDEN), jnp.float32),
        jax.ShapeDtypeStruct((BATCH,), jnp.float32),
        jax.ShapeDtypeStruct((BATCH,), jnp.float32),
        jax.ShapeDtypeStruct((BATCH,), jnp.float32),
    ),
    scratch_types=(
        pltpu.VMEM((_BW,), jnp.int32),
        pltpu.VMEM((_BW,), jnp.int32),
        pltpu.VMEM((_BW,), jnp.int32),
        pltpu.VMEM((_BW,), jnp.int32),
        pltpu.VMEM((_BW,), jnp.int32),
        pltpu.VMEM((_BW,), jnp.int32),
        pltpu.VMEM((_BW,), jnp.int32),
        pltpu.VMEM((_BW,), jnp.int32),
        pltpu.VMEM((_BW, 128), jnp.float32),
        pltpu.VMEM((_BW, 128), jnp.float32),
        pltpu.VMEM((_BW, HIDDEN), jnp.float32),
        pltpu.VMEM((_BW, HIDDEN), jnp.float32),
        pltpu.VMEM((_BW, HIDDEN), jnp.float32),
        pltpu.VMEM((_BW, HIDDEN), jnp.float32),
        pltpu.VMEM((_BW, HIDDEN), jnp.float32),
        pltpu.VMEM((_BW,), jnp.float32),
        pltpu.VMEM((_BW,), jnp.float32),
        pltpu.VMEM((_BW,), jnp.float32),
        pltpu.VMEM((_BW,), jnp.float32),
        pltpu.SemaphoreType.DMA,
        pltpu.SemaphoreType.DMA,
        pltpu.SemaphoreType.DMA,
        pltpu.SemaphoreType.DMA,
    ),
)
def _gather_all(ent2, rel2, diag, curv1, ctx2, hb1, tb1, head, tail, rel, neg,
                head_o, tail_o, rel_o, neg_o, curv_o, diag_o, ctx_o,
                hb_o, tb_o, ntb_o,
                hp, tp, rp, np_, hidx, tidx, ridx, nidx,
                lineA, lineB,
                hrow, trow, rrow, nrow, xrow, crow,
                hbrow, tbrow, ntbrow,
                gsem, semA, semB, ssem):
    wid = lax.axis_index("s") * _NC + lax.axis_index("c")
    base = wid * _BW
    sl = pl.ds(base, _BW)

    pltpu.sync_copy(head.at[sl], hidx)
    pltpu.sync_copy(tail.at[sl], tidx)
    pltpu.sync_copy(rel.at[sl], ridx)
    pltpu.sync_copy(neg.at[sl], nidx)

    def mkpairs(i, _):
        s16 = pl.ds(i * 16, 16)
        hp[s16] = hidx[s16] >> 1
        tp[s16] = tidx[s16] >> 1
        rp[s16] = ridx[s16] >> 1
        np_[s16] = nidx[s16] >> 1
        return 0
    lax.fori_loop(0, _BW // 16, mkpairs, 0, unroll=True)

    small = [
        pltpu.async_copy(curv1.at[ridx], crow, gsem),
        pltpu.async_copy(hb1.at[hidx], hbrow, gsem),
        pltpu.async_copy(tb1.at[tidx], tbrow, gsem),
        pltpu.async_copy(tb1.at[nidx], ntbrow, gsem),
    ]

    iota16 = lax.iota(jnp.int32, 16)

    def sel_table(line, idxv, row):
        for k in range(_BW // 16):
            hvec = iota16 + (16 * k)
            colb = (idxv[pl.ds(16 * k, 16)] & 1) * HIDDEN

            def cbody(c, _):
                v = plsc.load_gather(line, [hvec, colb + c])
                cvec = jnp.full((16,), c, dtype=jnp.int32)
                plsc.store_scatter(row, [hvec, cvec], v)
                return 0
            lax.fori_loop(0, HIDDEN, cbody, 0)

    d0 = pltpu.async_copy(ent2.at[hp], lineA, semA)
    d1 = pltpu.async_copy(ent2.at[tp], lineB, semB)
    stores = []
    d0.wait()
    sel_table(lineA, hidx, hrow)
    d2 = pltpu.async_copy(ent2.at[np_], lineA, semA)
    stores.append(pltpu.async_copy(hrow, head_o.at[sl], ssem))
    d1.wait()
    sel_table(lineB, tidx, trow)
    d3 = pltpu.async_copy(rel2.at[rp], lineB, semB)
    stores.append(pltpu.async_copy(trow, tail_o.at[sl], ssem))
    d2.wait()
    sel_table(lineA, nidx, nrow)
    d4 = pltpu.async_copy(ctx2.at[rp], lineA, semA)
    stores.append(pltpu.async_copy(nrow, neg_o.at[sl], ssem))
    d3.wait()
    sel_table(lineB, ridx, rrow)
    dd = pltpu.async_copy(diag.at[ridx], lineB, semB)
    stores.append(pltpu.async_copy(rrow, rel_o.at[sl], ssem))
    d4.wait()
    sel_table(lineA, ridx, xrow)
    stores.append(pltpu.async_copy(xrow, ctx_o.at[sl], ssem))

    for g in small:
        g.wait()
    dd.wait()
    stores += [
        pltpu.async_copy(crow, curv_o.at[sl], ssem),
        pltpu.async_copy(lineB, diag_o.at[sl], ssem),
        pltpu.async_copy(hbrow, hb_o.at[sl], ssem),
        pltpu.async_copy(tbrow, tb_o.at[sl], ssem),
        pltpu.async_copy(ntbrow, ntb_o.at[sl], ssem),
    ]
    for s in stores:
        s.wait()


def kernel(entity_emb, relation_emb, relation_diag, curvature, context,
           head_bias, tail_bias, head, tail, rel, neg):
    scale = jnp.array([0.125], dtype=jnp.float32)
    (head_e, tail_e, rel_e, neg_e, curv, rel_diag, ctx,
     h_bias, t_bias, neg_t_bias) = _gather_all(
        entity_emb.reshape(N_ENTITY // 2, 2 * HIDDEN),
        relation_emb.reshape(N_RELATION // 2, 2 * HIDDEN),
        relation_diag,
        curvature.reshape(N_RELATION),
        context.reshape(N_RELATION // 2, 2 * HIDDEN),
        head_bias.reshape(N_ENTITY), tail_bias.reshape(N_ENTITY),
        head.astype(jnp.int32), tail.astype(jnp.int32),
        rel.astype(jnp.int32), neg.astype(jnp.int32))
    return (scale, head_e, tail_e, rel_e, neg_e,
            curv.reshape(BATCH, 1), rel_diag, ctx,
            h_bias.reshape(BATCH, 1), t_bias.reshape(BATCH, 1),
            neg_t_bias.reshape(BATCH, 1))


# speedup vs baseline: 1.0775x; 1.0775x over previous
import functools

import jax
import jax.numpy as jnp
from jax import lax
from jax.experimental import pallas as pl
from jax.experimental.pallas import tpu as pltpu
from jax.experimental.pallas import tpu_sc as plsc

N_ENTITY = 1000000
N_RELATION = 1000
HIDDEN = 64
BATCH = 4096

_NC, _NS = 2, 16
_NW = _NC * _NS
_BW = BATCH // _NW
_NTC = (N_ENTITY + 127) // 128
_KCAP = 3 * BATCH
_RING = 128

_mesh = plsc.VectorSubcoreMesh(core_axis_name="c", subcore_axis_name="s")


@functools.partial(
    pl.kernel,
    mesh=_mesh,
    compiler_params=pltpu.CompilerParams(use_tc_tiling_on_sc=True,
                                         needs_layout_passes=False),
    out_type=(
        jax.ShapeDtypeStruct((BATCH * HIDDEN,), jnp.float32),
        jax.ShapeDtypeStruct((BATCH * HIDDEN,), jnp.float32),
        jax.ShapeDtypeStruct((BATCH, HIDDEN), jnp.float32),
        jax.ShapeDtypeStruct((BATCH * HIDDEN,), jnp.float32),
        jax.ShapeDtypeStruct((BATCH,), jnp.float32),
        jax.ShapeDtypeStruct((BATCH, 2 * HIDDEN), jnp.float32),
        jax.ShapeDtypeStruct((BATCH, HIDDEN), jnp.float32),
        jax.ShapeDtypeStruct((BATCH,), jnp.float32),
        jax.ShapeDtypeStruct((BATCH,), jnp.float32),
        jax.ShapeDtypeStruct((BATCH,), jnp.float32),
    ),
    scratch_types=(
        pltpu.VMEM((_BW,), jnp.int32),
        pltpu.VMEM((_BW,), jnp.int32),
        pltpu.VMEM((_BW,), jnp.int32),
        pltpu.VMEM((_BW,), jnp.int32),
        pltpu.VMEM((_BW,), jnp.int32),
        pltpu.VMEM((BATCH,), jnp.int32),
        pltpu.VMEM((BATCH,), jnp.int32),
        pltpu.VMEM((BATCH,), jnp.int32),
        pltpu.VMEM((_KCAP,), jnp.int32),
        pltpu.VMEM((64, 128), jnp.float32),
        pltpu.VMEM((64, 128), jnp.float32),
        pltpu.VMEM((_RING, HIDDEN), jnp.float32),
        pltpu.VMEM((_BW, 128), jnp.float32),
        pltpu.VMEM((_BW, 128), jnp.float32),
        pltpu.VMEM((_BW, HIDDEN), jnp.float32),
        pltpu.VMEM((_BW, HIDDEN), jnp.float32),
        pltpu.VMEM((_BW,), jnp.float32),
        pltpu.VMEM((_BW,), jnp.float32),
        pltpu.VMEM((_BW,), jnp.float32),
        pltpu.VMEM((_BW,), jnp.float32),
        pltpu.SemaphoreType.DMA,
        pltpu.SemaphoreType.DMA,
        pltpu.SemaphoreType.DMA,
        pltpu.SemaphoreType.DMA,
        pltpu.SemaphoreType.DMA,
        pltpu.SemaphoreType.DMA,
        pltpu.SemaphoreType.DMA,
    ),
)
def _gather_all(etT, rel2, diag, curv1, ctx2, hb1, tb1, head, tail, rel, neg,
                head_o, tail_o, rel_o, neg_o, curv_o, diag_o, ctx_o,
                hb_o, tb_o, ntb_o,
                ridx, rp, hidx, tidx, nidx, hfull, tfull, nfull,
                hits, chunkA, chunkB, ring,
                lineA, lineB, rrow, xrow, crow, hbrow, tbrow, ntbrow,
                gsem, semA, semB, csemA, csemB, rowsem, ssem):
    wid = lax.axis_index("s") * _NC + lax.axis_index("c")
    base = wid * _BW
    sl = pl.ds(base, _BW)
    iota16 = lax.iota(jnp.int32, 16)

    pltpu.sync_copy(rel.at[sl], ridx)
    pltpu.sync_copy(head.at[sl], hidx)
    pltpu.sync_copy(tail.at[sl], tidx)
    pltpu.sync_copy(neg.at[sl], nidx)

    def mkpairs(i, _):
        s16 = pl.ds(i * 16, 16)
        rp[s16] = ridx[s16] >> 1
        return 0
    lax.fori_loop(0, _BW // 16, mkpairs, 0, unroll=True)

    small = [
        pltpu.async_copy(curv1.at[ridx], crow, gsem),
        pltpu.async_copy(hb1.at[hidx], hbrow, gsem),
        pltpu.async_copy(tb1.at[tidx], tbrow, gsem),
        pltpu.async_copy(tb1.at[nidx], ntbrow, gsem),
    ]
    dA = pltpu.async_copy(rel2.at[rp], lineA, semA)
    dB = pltpu.async_copy(ctx2.at[rp], lineB, semB)

    def sel_table(line, idxv, row):
        for k in range(_BW // 16):
            hvec = iota16 + (16 * k)
            colb = (idxv[pl.ds(16 * k, 16)] & 1) * HIDDEN

            def cbody(c, _):
                v = plsc.load_gather(line, [hvec, colb + c])
                cvec = jnp.full((16,), c, dtype=jnp.int32)
                plsc.store_scatter(row, [hvec, cvec], v)
                return 0
            lax.fori_loop(0, HIDDEN, cbody, 0)

    dA.wait()
    sel_table(lineA, ridx, rrow)
    dD = pltpu.async_copy(diag.at[ridx], lineA, semA)
    st_rel = pltpu.async_copy(rrow, rel_o.at[sl], ssem)
    dB.wait()
    sel_table(lineB, ridx, xrow)
    st_ctx = pltpu.async_copy(xrow, ctx_o.at[sl], ssem)

    pltpu.sync_copy(head, hfull)
    pltpu.sync_copy(tail, tfull)
    pltpu.sync_copy(neg, nfull)

    t0 = (wid * _NTC) // _NW
    t1 = ((wid + 1) * _NTC) // _NW

    def build(tblref, tblid, kcnt0):
        def chunk(j, kcnt):
            r = tblref[pl.ds(j * 16, 16)]
            tc = r >> 7
            m = (tc >= t0) & (tc < t1)
            n = lax.reduce_max(plsc.all_reduce_population_count(m), (0,))

            def have():
                pos = kcnt + plsc.cumsum(m.astype(jnp.int32)) - 1
                ea = ((tc - t0) << 7) | (r & 127)
                eb = (16 * j + iota16) | (tblid << 12)
                plsc.store_scatter(hits, [pos], ea | (eb << 15), mask=m)
            pl.when(n > 0)(have)
            return kcnt + n
        return lax.fori_loop(0, BATCH // 16, chunk, kcnt0)

    kcnt = build(hfull, 0, jnp.int32(0))
    kcnt = build(tfull, 1, kcnt)
    kcnt = build(nfull, 2, kcnt)
    kchunks = (kcnt + 15) >> 4

    ncols = t1 - t0
    dummy = hb1.at[pl.ds(0, HIDDEN)]

    c0 = pltpu.async_copy(etT.at[:, pl.ds(t0 * 128, 128)], chunkA, csemA)

    def process_col(trel, chunk, state):
        issued, drained = state

        def kchunk(j, st):
            iss, drn = st
            a = hits[pl.ds(j * 16, 16)]
            m = (((a & 0x7FFF) >> 7) == trel) & ((j * 16 + iota16) < kcnt)
            n = lax.reduce_max(plsc.all_reduce_population_count(m), (0,))

            def have(st2):
                iss2, drn2 = st2
                lvec = a & 127
                pos = plsc.cumsum(m.astype(jnp.int32)) - 1
                slot = (iss2 + pos) & (_RING - 1)

                def cbody(c, _):
                    cvec = jnp.full((16,), c, dtype=jnp.int32)
                    v = plsc.load_gather(chunk, [cvec, lvec], mask=m)
                    plsc.store_scatter(ring, [slot, cvec], v, mask=m)
                    return 0
                lax.fori_loop(0, HIDDEN, cbody, 0)
                eb = a >> 15
                mint = m.astype(jnp.int32)
                for i in range(16):
                    mi = mint[i] > 0
                    e = eb[i]
                    b = e & 4095
                    tbl = e >> 12
                    slot = (iss2 + pos[i]) & (_RING - 1)
                    src = ring.at[slot]
                    dst = pl.ds(b * HIDDEN, HIDDEN)

                    @pl.when(mi & (tbl == 0))
                    def _():
                        pltpu.async_copy(src, head_o.at[dst], rowsem)

                    @pl.when(mi & (tbl == 1))
                    def _():
                        pltpu.async_copy(src, tail_o.at[dst], rowsem)

                    @pl.when(mi & (tbl == 2))
                    def _():
                        pltpu.async_copy(src, neg_o.at[dst], rowsem)
                iss2 = iss2 + n

                def drain_some(st4):
                    iss4, drn4 = st4

                    def dr(i, d):
                        pltpu.make_async_copy(dummy, ring.at[0], rowsem).wait()
                        return d + 1
                    drn4 = lax.fori_loop(0, 64, dr, drn4)
                    return (iss4, drn4)
                return lax.cond(iss2 - drn2 >= _RING - 32, drain_some,
                                lambda s: s, (iss2, drn2))
            return lax.cond(n > 0, have, lambda s: s, (iss, drn))
        return lax.fori_loop(0, kchunks, kchunk, (issued, drained))

    nsteps = (_NTC // _NW) // 2 + 2

    def colpair(i, state):
        tA = t0 + 2 * i
        tB = tA + 1

        @pl.when(tB < t1)
        def _():
            pltpu.async_copy(etT.at[:, pl.ds(tB * 128, 128)], chunkB, csemB)

        def doA(st):
            pltpu.make_async_copy(etT.at[:, pl.ds(0, 128)], chunkA,
                                  csemA).wait()
            st = process_col(tA - t0, chunkA, st)

            @pl.when(tB + 1 < t1)
            def _():
                pltpu.async_copy(etT.at[:, pl.ds((tB + 1) * 128, 128)],
                                 chunkA, csemA)
            return st
        state = lax.cond(tA < t1, doA, lambda s: s, state)

        def doB(st):
            pltpu.make_async_copy(etT.at[:, pl.ds(0, 128)], chunkB,
                                  csemB).wait()
            return process_col(tB - t0, chunkB, st)
        return lax.cond(tB < t1, doB, lambda s: s, state)

    issued, drained = lax.fori_loop(0, nsteps, colpair,
                                    (jnp.int32(0), jnp.int32(0)))

    def drfin(i, d):
        pltpu.make_async_copy(dummy, ring.at[0], rowsem).wait()
        return d + 1
    lax.fori_loop(0, issued - drained, drfin, drained)

    for g in small:
        g.wait()
    dD.wait()
    stores = [
        st_rel, st_ctx,
        pltpu.async_copy(crow, curv_o.at[sl], ssem),
        pltpu.async_copy(lineA, diag_o.at[sl], ssem),
        pltpu.async_copy(hbrow, hb_o.at[sl], ssem),
        pltpu.async_copy(tbrow, tb_o.at[sl], ssem),
        pltpu.async_copy(ntbrow, ntb_o.at[sl], ssem),
    ]
    for s in stores:
        s.wait()


def kernel(entity_emb, relation_emb, relation_diag, curvature, context,
           head_bias, tail_bias, head, tail, rel, neg):
    scale = jnp.array([0.125], dtype=jnp.float32)
    (head_e, tail_e, rel_e, neg_e, curv, rel_diag, ctx,
     h_bias, t_bias, neg_t_bias) = _gather_all(
        entity_emb.T,
        relation_emb.reshape(N_RELATION // 2, 2 * HIDDEN),
        relation_diag,
        curvature.reshape(N_RELATION),
        context.reshape(N_RELATION // 2, 2 * HIDDEN),
        head_bias.reshape(N_ENTITY), tail_bias.reshape(N_ENTITY),
        head.astype(jnp.int32), tail.astype(jnp.int32),
        rel.astype(jnp.int32), neg.astype(jnp.int32))
    return (scale, head_e.reshape(BATCH, HIDDEN), tail_e.reshape(BATCH, HIDDEN),
            rel_e, neg_e.reshape(BATCH, HIDDEN),
            curv.reshape(BATCH, 1), rel_diag, ctx,
            h_bias.reshape(BATCH, 1), t_bias.reshape(BATCH, 1),
            neg_t_bias.reshape(BATCH, 1))


# speedup vs baseline: 1.1787x; 1.0939x over previous
import functools

import jax
import jax.numpy as jnp
from jax import lax
from jax.experimental import pallas as pl
from jax.experimental.pallas import tpu as pltpu
from jax.experimental.pallas import tpu_sc as plsc

N_ENTITY = 1000000
N_RELATION = 1000
HIDDEN = 64
BATCH = 4096

_NC, _NS = 2, 16
_NW = _NC * _NS
_BW = BATCH // _NW
_NTC = (N_ENTITY + 127) // 128
_KCAP = 3 * BATCH
_RING = 64

_mesh = plsc.VectorSubcoreMesh(core_axis_name="c", subcore_axis_name="s")


@functools.partial(
    pl.kernel,
    mesh=_mesh,
    compiler_params=pltpu.CompilerParams(use_tc_tiling_on_sc=True,
                                         needs_layout_passes=False),
    out_type=(
        jax.ShapeDtypeStruct((BATCH * HIDDEN,), jnp.float32),
        jax.ShapeDtypeStruct((BATCH * HIDDEN,), jnp.float32),
        jax.ShapeDtypeStruct((BATCH, HIDDEN), jnp.float32),
        jax.ShapeDtypeStruct((BATCH * HIDDEN,), jnp.float32),
        jax.ShapeDtypeStruct((BATCH,), jnp.float32),
        jax.ShapeDtypeStruct((BATCH, 2 * HIDDEN), jnp.float32),
        jax.ShapeDtypeStruct((BATCH, HIDDEN), jnp.float32),
        jax.ShapeDtypeStruct((BATCH,), jnp.float32),
        jax.ShapeDtypeStruct((BATCH,), jnp.float32),
        jax.ShapeDtypeStruct((BATCH,), jnp.float32),
    ),
    scratch_types=(
        pltpu.VMEM((_BW,), jnp.int32),
        pltpu.VMEM((_BW,), jnp.int32),
        pltpu.VMEM((_BW,), jnp.int32),
        pltpu.VMEM((_BW,), jnp.int32),
        pltpu.VMEM((_BW,), jnp.int32),
        pltpu.VMEM((BATCH,), jnp.int32),
        pltpu.VMEM((BATCH,), jnp.int32),
        pltpu.VMEM((BATCH,), jnp.int32),
        pltpu.VMEM((_KCAP,), jnp.int32),
        pltpu.VMEM((_KCAP,), jnp.int32),
        pltpu.SMEM((16,), jnp.int32),
        pltpu.SMEM((16,), jnp.int32),
        pltpu.SMEM((256,), jnp.int32),
        pltpu.VMEM((64, 128), jnp.float32),
        pltpu.VMEM((64, 128), jnp.float32),
        pltpu.VMEM((64, 128), jnp.float32),
        pltpu.VMEM((64, 128), jnp.float32),
        pltpu.VMEM((_RING, HIDDEN), jnp.float32),
        pltpu.VMEM((_BW, 128), jnp.float32),
        pltpu.VMEM((_BW, HIDDEN), jnp.float32),
        pltpu.VMEM((_BW, HIDDEN), jnp.float32),
        pltpu.VMEM((_BW,), jnp.float32),
        pltpu.VMEM((_BW,), jnp.float32),
        pltpu.VMEM((_BW,), jnp.float32),
        pltpu.VMEM((_BW,), jnp.float32),
        pltpu.SemaphoreType.DMA,
        pltpu.SemaphoreType.DMA,
        pltpu.SemaphoreType.DMA,
        pltpu.SemaphoreType.DMA,
        pltpu.SemaphoreType.DMA,
        pltpu.SemaphoreType.DMA,
        pltpu.SemaphoreType.DMA,
        pltpu.SemaphoreType.DMA,
    ),
)
def _gather_all(etT, rel2, diag, curv1, ctx2, hb1, tb1, head, tail, rel, neg,
                head_o, tail_o, rel_o, neg_o, curv_o, diag_o, ctx_o,
                hb_o, tb_o, ntb_o,
                ridx, rp, hidx, tidx, nidx, hfull, tfull, nfull,
                hits, hits2, histsm, basesm, colsm, chunk0, chunk1, chunk2,
                chunk3, ring,
                lineA, rrow, xrow, crow, hbrow, tbrow, ntbrow,
                gsem, semA, csem0, csem1, csem2, csem3, rowsem, ssem):
    wid = lax.axis_index("s") * _NC + lax.axis_index("c")
    base = wid * _BW
    sl = pl.ds(base, _BW)
    iota16 = lax.iota(jnp.int32, 16)

    pltpu.sync_copy(rel.at[sl], ridx)
    pltpu.sync_copy(head.at[sl], hidx)
    pltpu.sync_copy(tail.at[sl], tidx)
    pltpu.sync_copy(neg.at[sl], nidx)

    def mkpairs(i, _):
        s16 = pl.ds(i * 16, 16)
        rp[s16] = ridx[s16] >> 1
        return 0
    lax.fori_loop(0, _BW // 16, mkpairs, 0, unroll=True)

    small = [
        pltpu.async_copy(curv1.at[ridx], crow, gsem),
        pltpu.async_copy(hb1.at[hidx], hbrow, gsem),
        pltpu.async_copy(tb1.at[tidx], tbrow, gsem),
        pltpu.async_copy(tb1.at[nidx], ntbrow, gsem),
    ]
    dA = pltpu.async_copy(rel2.at[rp], lineA, semA)

    def sel_table(line, idxv, row):
        for k in range(_BW // 16):
            hvec = iota16 + (16 * k)
            colb = (idxv[pl.ds(16 * k, 16)] & 1) * HIDDEN

            def cbody(c, _):
                v = plsc.load_gather(line, [hvec, colb + c])
                cvec = jnp.full((16,), c, dtype=jnp.int32)
                plsc.store_scatter(row, [hvec, cvec], v)
                return 0
            lax.fori_loop(0, HIDDEN, cbody, 0)

    pltpu.sync_copy(head, hfull)
    pltpu.sync_copy(tail, tfull)
    pltpu.sync_copy(neg, nfull)

    t0 = (wid * _NTC) // _NW
    t1 = ((wid + 1) * _NTC) // _NW

    def build(tblref, tblid, kcnt0):
        def chunk(j, kcnt):
            r = tblref[pl.ds(j * 16, 16)]
            tc = r >> 7
            m = (tc >= t0) & (tc < t1)
            n = lax.reduce_max(plsc.all_reduce_population_count(m), (0,))

            def have():
                pos = kcnt + plsc.cumsum(m.astype(jnp.int32)) - 1
                ea = ((tc - t0) << 7) | (r & 127)
                eb = (16 * j + iota16) | (tblid << 12)
                plsc.store_scatter(hits, [pos], ea | (eb << 15), mask=m)
            pl.when(n > 0)(have)
            return kcnt + n
        return lax.fori_loop(0, BATCH // 16, chunk, kcnt0)

    kcnt = build(hfull, 0, jnp.int32(0))
    kcnt = build(tfull, 1, kcnt)
    kcnt = build(nfull, 2, kcnt)
    kchunks = (kcnt + 15) >> 4

    ncols = t1 - t0
    dummy = hb1.at[pl.ds(0, HIDDEN)]
    zeros16 = jnp.zeros((16,), jnp.int32)

    def subs_of(a):
        return (a & 0x7FFF) >> 11

    def count_chunk(j, hist):
        a = hits[pl.ds(j * 16, 16)]
        valid = ((j * 16 + iota16) < kcnt).astype(jnp.int32)
        sub = subs_of(a)
        for i in range(16):
            hist = hist + (iota16 == sub[i]).astype(jnp.int32) * valid[i]
        return hist
    hist = lax.fori_loop(0, kchunks, count_chunk, zeros16)
    basevec = plsc.cumsum(hist) - hist

    def zerocol(i, _):
        colsm[i] = 0
        return 0
    lax.fori_loop(0, 256, zerocol, 0)

    def place_chunk(j, runvec):
        a = hits[pl.ds(j * 16, 16)]
        validb = (j * 16 + iota16) < kcnt
        valid = validb.astype(jnp.int32)
        sub = subs_of(a)
        rank = zeros16
        hcnt = zeros16
        bpr = basevec + runvec
        pb = zeros16
        for i in range(16):
            rank = rank + ((sub == sub[i]) & (iota16 > i)).astype(
                jnp.int32) * valid[i]
            hcnt = hcnt + (iota16 == sub[i]).astype(jnp.int32) * valid[i]
            pb = pb + (sub == i).astype(jnp.int32) * bpr[i]
        pos = pb + rank
        plsc.store_scatter(hits2, [pos], a, mask=validb)
        trel = (a & 0x7FFF) >> 7
        for i in range(16):
            @pl.when(valid[i] > 0)
            def _():
                colsm[trel[i]] = colsm[trel[i]] + 1
        return runvec + hcnt
    lax.fori_loop(0, kchunks, place_chunk, zeros16)
    for s in range(16):
        histsm[s] = hist[s]
        basesm[s] = basevec[s]

    dA.wait()
    sel_table(lineA, ridx, rrow)
    dX = pltpu.async_copy(ctx2.at[rp], lineA, semA)
    st_rel = pltpu.async_copy(rrow, rel_o.at[sl], ssem)
    dX.wait()
    sel_table(lineA, ridx, xrow)
    dD = pltpu.async_copy(diag.at[ridx], lineA, semA)
    st_ctx = pltpu.async_copy(xrow, ctx_o.at[sl], ssem)

    def process_col(trel, seg_base, seg_cnt, chunk, state):
        def kchunk(j, st):
            a = hits2[pl.ds(seg_base + j * 16, 16)]
            m = (((a & 0x7FFF) >> 7) == trel) & ((j * 16 + iota16) < seg_cnt)
            n = lax.reduce_max(plsc.all_reduce_population_count(m), (0,))

            def have(st2):
                iss2, drn2 = st2
                lv = a & 127
                eb = a >> 15
                mint = m.astype(jnp.int32)
                cur = iss2
                for i in range(16):
                    mi = mint[i] > 0
                    e = eb[i]
                    b = e & 4095
                    tbl = e >> 12
                    slot = cur & (_RING - 1)

                    @pl.when(mi)
                    def _():
                        lbc = jnp.full((16,), lv[i], dtype=jnp.int32)
                        for q in range(4):
                            v = plsc.load_gather(chunk, [iota16 + 16 * q, lbc])
                            ring[slot, pl.ds(16 * q, 16)] = v
                    src = ring.at[slot]
                    dst = pl.ds(b * HIDDEN, HIDDEN)

                    @pl.when(mi & (tbl == 0))
                    def _():
                        pltpu.async_copy(src, head_o.at[dst], rowsem)

                    @pl.when(mi & (tbl == 1))
                    def _():
                        pltpu.async_copy(src, tail_o.at[dst], rowsem)

                    @pl.when(mi & (tbl == 2))
                    def _():
                        pltpu.async_copy(src, neg_o.at[dst], rowsem)
                    cur = cur + mint[i]
                iss2 = cur

                def drain_some(st4):
                    iss4, drn4 = st4

                    def dr(i2, d):
                        pltpu.make_async_copy(dummy, ring.at[0], rowsem).wait()
                        return d + 1
                    drn4 = lax.fori_loop(0, 16, dr, drn4)
                    return (iss4, drn4)
                return lax.cond(iss2 - drn2 >= _RING - 32, drain_some,
                                lambda s: s, (iss2, drn2))
            return lax.cond(n > 0, have, lambda s: s, st)
        return lax.fori_loop(0, (seg_cnt + 15) >> 4, kchunk, state)

    def subbody(s, state):
        seg_cnt = histsm[s]
        seg_base = basesm[s]
        colbase = 16 * s
        climit = jnp.minimum(colbase + 16, ncols)
        go = seg_cnt > 0

        bufs = (chunk0, chunk1, chunk2, chunk3)
        sems = (csem0, csem1, csem2, csem3)

        def want(c):
            return go & (c < climit) & (colsm[jnp.minimum(c, 255)] > 0)

        def fire(c, q):
            @pl.when(want(c))
            def _():
                pltpu.async_copy(etT.at[:, pl.ds((t0 + c) * 128, 128)],
                                 bufs[q], sems[q])

        for q in range(3):
            fire(colbase + q, q)

        def quadbody(i, st):
            for q in range(4):
                c = colbase + 4 * i + q
                fire(c + 3, (q + 3) % 4)

                def do(st2, c=c, q=q):
                    pltpu.make_async_copy(etT.at[:, pl.ds(0, 128)], bufs[q],
                                          sems[q]).wait()
                    return process_col(c, seg_base, seg_cnt, bufs[q], st2)
                st = lax.cond(want(c), do, lambda s2: s2, st)
            return st

        return lax.fori_loop(0, 4, quadbody, state)

    issued, drained = lax.fori_loop(0, 16, subbody,
                                    (jnp.int32(0), jnp.int32(0)))

    def drfin(i, d):
        pltpu.make_async_copy(dummy, ring.at[0], rowsem).wait()
        return d + 1
    lax.fori_loop(0, issued - drained, drfin, drained)

    for g in small:
        g.wait()
    dD.wait()
    stores = [
        st_rel, st_ctx,
        pltpu.async_copy(crow, curv_o.at[sl], ssem),
        pltpu.async_copy(lineA, diag_o.at[sl], ssem),
        pltpu.async_copy(hbrow, hb_o.at[sl], ssem),
        pltpu.async_copy(tbrow, tb_o.at[sl], ssem),
        pltpu.async_copy(ntbrow, ntb_o.at[sl], ssem),
    ]
    for s in stores:
        s.wait()


def kernel(entity_emb, relation_emb, relation_diag, curvature, context,
           head_bias, tail_bias, head, tail, rel, neg):
    scale = jnp.array([0.125], dtype=jnp.float32)
    (head_e, tail_e, rel_e, neg_e, curv, rel_diag, ctx,
     h_bias, t_bias, neg_t_bias) = _gather_all(
        entity_emb.T,
        relation_emb.reshape(N_RELATION // 2, 2 * HIDDEN),
        relation_diag,
        curvature.reshape(N_RELATION),
        context.reshape(N_RELATION // 2, 2 * HIDDEN),
        head_bias.reshape(N_ENTITY), tail_bias.reshape(N_ENTITY),
        head.astype(jnp.int32), tail.astype(jnp.int32),
        rel.astype(jnp.int32), neg.astype(jnp.int32))
    return (scale, head_e.reshape(BATCH, HIDDEN), tail_e.reshape(BATCH, HIDDEN),
            rel_e, neg_e.reshape(BATCH, HIDDEN),
            curv.reshape(BATCH, 1), rel_diag, ctx,
            h_bias.reshape(BATCH, 1), t_bias.reshape(BATCH, 1),
            neg_t_bias.reshape(BATCH, 1))


# speedup vs baseline: 1.3452x; 1.1413x over previous
import functools

import jax
import jax.numpy as jnp
from jax import lax
from jax.experimental import pallas as pl
from jax.experimental.pallas import tpu as pltpu
from jax.experimental.pallas import tpu_sc as plsc

N_ENTITY = 1000000
N_RELATION = 1000
HIDDEN = 64
BATCH = 4096

_NC, _NS = 2, 16
_NW = _NC * _NS
_BW = BATCH // _NW
_NTC = (N_ENTITY + 127) // 128
_KCAP = 3 * BATCH
_RING = 64

_mesh = plsc.VectorSubcoreMesh(core_axis_name="c", subcore_axis_name="s")


@functools.partial(
    pl.kernel,
    mesh=_mesh,
    compiler_params=pltpu.CompilerParams(use_tc_tiling_on_sc=True,
                                         needs_layout_passes=False),
    out_type=(
        jax.ShapeDtypeStruct((BATCH * HIDDEN,), jnp.float32),
        jax.ShapeDtypeStruct((BATCH * HIDDEN,), jnp.float32),
        jax.ShapeDtypeStruct((BATCH, HIDDEN), jnp.float32),
        jax.ShapeDtypeStruct((BATCH * HIDDEN,), jnp.float32),
        jax.ShapeDtypeStruct((BATCH,), jnp.float32),
        jax.ShapeDtypeStruct((BATCH, 2 * HIDDEN), jnp.float32),
        jax.ShapeDtypeStruct((BATCH, HIDDEN), jnp.float32),
        jax.ShapeDtypeStruct((BATCH,), jnp.float32),
        jax.ShapeDtypeStruct((BATCH,), jnp.float32),
        jax.ShapeDtypeStruct((BATCH,), jnp.float32),
    ),
    scratch_types=(
        pltpu.VMEM((_BW,), jnp.int32),
        pltpu.VMEM((_BW,), jnp.int32),
        pltpu.VMEM((_BW,), jnp.int32),
        pltpu.VMEM((_BW,), jnp.int32),
        pltpu.VMEM((_BW,), jnp.int32),
        pltpu.VMEM((BATCH,), jnp.int32),
        pltpu.VMEM((BATCH,), jnp.int32),
        pltpu.VMEM((BATCH,), jnp.int32),
        pltpu.VMEM((_KCAP,), jnp.int32),
        pltpu.VMEM((_KCAP,), jnp.int32),
        pltpu.SMEM((256,), jnp.int32),
        pltpu.SMEM((256,), jnp.int32),
        pltpu.SMEM((256,), jnp.int32),
        pltpu.VMEM((64, 128), jnp.float32),
        pltpu.VMEM((64, 128), jnp.float32),
        pltpu.VMEM((64, 128), jnp.float32),
        pltpu.VMEM((64, 128), jnp.float32),
        pltpu.VMEM((_RING, HIDDEN), jnp.float32),
        pltpu.VMEM((_BW, 128), jnp.float32),
        pltpu.VMEM((_BW, HIDDEN), jnp.float32),
        pltpu.VMEM((_BW, HIDDEN), jnp.float32),
        pltpu.VMEM((_BW,), jnp.float32),
        pltpu.VMEM((_BW,), jnp.float32),
        pltpu.VMEM((_BW,), jnp.float32),
        pltpu.VMEM((_BW,), jnp.float32),
        pltpu.SemaphoreType.DMA,
        pltpu.SemaphoreType.DMA,
        pltpu.SemaphoreType.DMA,
        pltpu.SemaphoreType.DMA,
        pltpu.SemaphoreType.DMA,
        pltpu.SemaphoreType.DMA,
        pltpu.SemaphoreType.DMA,
        pltpu.SemaphoreType.DMA,
    ),
)
def _gather_all(etT, rel2, diag, curv1, ctx2, hb1, tb1, head, tail, rel, neg,
                head_o, tail_o, rel_o, neg_o, curv_o, diag_o, ctx_o,
                hb_o, tb_o, ntb_o,
                ridx, rp, hidx, tidx, nidx, hfull, tfull, nfull,
                hits, hits2, colsm, colstart, coloff, chunk0, chunk1, chunk2,
                chunk3, ring,
                lineA, rrow, xrow, crow, hbrow, tbrow, ntbrow,
                gsem, semA, csem0, csem1, csem2, csem3, rowsem, ssem):
    wid = lax.axis_index("s") * _NC + lax.axis_index("c")
    base = wid * _BW
    sl = pl.ds(base, _BW)
    iota16 = lax.iota(jnp.int32, 16)

    pltpu.sync_copy(rel.at[sl], ridx)
    pltpu.sync_copy(head.at[sl], hidx)
    pltpu.sync_copy(tail.at[sl], tidx)
    pltpu.sync_copy(neg.at[sl], nidx)

    def mkpairs(i, _):
        s16 = pl.ds(i * 16, 16)
        rp[s16] = ridx[s16] >> 1
        return 0
    lax.fori_loop(0, _BW // 16, mkpairs, 0, unroll=True)

    small = [
        pltpu.async_copy(curv1.at[ridx], crow, gsem),
        pltpu.async_copy(hb1.at[hidx], hbrow, gsem),
        pltpu.async_copy(tb1.at[tidx], tbrow, gsem),
        pltpu.async_copy(tb1.at[nidx], ntbrow, gsem),
    ]
    dA = pltpu.async_copy(rel2.at[rp], lineA, semA)

    def sel_table(line, idxv, row):
        for k in range(_BW // 16):
            hvec = iota16 + (16 * k)
            colb = (idxv[pl.ds(16 * k, 16)] & 1) * HIDDEN

            def cbody(c, _):
                v = plsc.load_gather(line, [hvec, colb + c])
                cvec = jnp.full((16,), c, dtype=jnp.int32)
                plsc.store_scatter(row, [hvec, cvec], v)
                return 0
            lax.fori_loop(0, HIDDEN, cbody, 0)

    pltpu.sync_copy(head, hfull)
    pltpu.sync_copy(tail, tfull)
    pltpu.sync_copy(neg, nfull)

    t0 = (wid * _NTC) // _NW
    t1 = ((wid + 1) * _NTC) // _NW

    def build(tblref, tblid, kcnt0):
        def chunk(j, kcnt):
            r = tblref[pl.ds(j * 16, 16)]
            tc = r >> 7
            m = (tc >= t0) & (tc < t1)
            n = lax.reduce_max(plsc.all_reduce_population_count(m), (0,))

            def have():
                pos = kcnt + plsc.cumsum(m.astype(jnp.int32)) - 1
                ea = ((tc - t0) << 7) | (r & 127)
                eb = (16 * j + iota16) | (tblid << 12)
                plsc.store_scatter(hits, [pos], ea | (eb << 15), mask=m)
            pl.when(n > 0)(have)
            return kcnt + n
        return lax.fori_loop(0, BATCH // 16, chunk, kcnt0)

    kcnt = build(hfull, 0, jnp.int32(0))
    kcnt = build(tfull, 1, kcnt)
    kcnt = build(nfull, 2, kcnt)
    kchunks = (kcnt + 15) >> 4

    ncols = t1 - t0
    dummy = hb1.at[pl.ds(0, HIDDEN)]
    zeros16 = jnp.zeros((16,), jnp.int32)

    def zerocol(i, _):
        colsm[i] = 0
        return 0
    lax.fori_loop(0, 256, zerocol, 0)

    def count_chunk(j, _):
        a = hits[pl.ds(j * 16, 16)]
        valid = ((j * 16 + iota16) < kcnt).astype(jnp.int32)
        trel = (a & 0x7FFF) >> 7
        for i in range(16):
            colsm[trel[i]] = colsm[trel[i]] + valid[i]
        return 0
    lax.fori_loop(0, kchunks, count_chunk, 0)

    def prefix(i, run):
        colstart[i] = run
        coloff[i] = run
        return run + colsm[i]
    lax.fori_loop(0, 256, prefix, jnp.int32(0))

    def place_chunk(j, _):
        a = hits[pl.ds(j * 16, 16)]
        validb = (j * 16 + iota16) < kcnt
        valid = validb.astype(jnp.int32)
        trel = (a & 0x7FFF) >> 7
        pos = zeros16
        for i in range(16):
            p = coloff[trel[i]]
            coloff[trel[i]] = p + valid[i]
            pos = jnp.where(iota16 == i, p, pos)
        plsc.store_scatter(hits2, [pos], a, mask=validb)
        return 0
    lax.fori_loop(0, kchunks, place_chunk, 0)

    dA.wait()
    sel_table(lineA, ridx, rrow)
    dX = pltpu.async_copy(ctx2.at[rp], lineA, semA)
    st_rel = pltpu.async_copy(rrow, rel_o.at[sl], ssem)
    dX.wait()
    sel_table(lineA, ridx, xrow)
    dD = pltpu.async_copy(diag.at[ridx], lineA, semA)
    st_ctx = pltpu.async_copy(xrow, ctx_o.at[sl], ssem)

    def process_col(seg_base, seg_cnt, chunk, state):
        def kchunk(j, st):
            a = hits2[pl.ds(seg_base + j * 16, 16)]
            m = (j * 16 + iota16) < seg_cnt
            n = jnp.minimum(seg_cnt - j * 16, 16)

            def have(st2):
                iss2, drn2 = st2
                lv = a & 127
                eb = a >> 15
                mint = m.astype(jnp.int32)
                cur = iss2
                for i in range(16):
                    mi = mint[i] > 0
                    e = eb[i]
                    b = e & 4095
                    tbl = e >> 12
                    slot = cur & (_RING - 1)

                    @pl.when(mi)
                    def _():
                        lbc = jnp.full((16,), lv[i], dtype=jnp.int32)
                        for q in range(4):
                            v = plsc.load_gather(chunk, [iota16 + 16 * q, lbc])
                            ring[slot, pl.ds(16 * q, 16)] = v
                    src = ring.at[slot]
                    dst = pl.ds(b * HIDDEN, HIDDEN)

                    @pl.when(mi & (tbl == 0))
                    def _():
                        pltpu.async_copy(src, head_o.at[dst], rowsem)

                    @pl.when(mi & (tbl == 1))
                    def _():
                        pltpu.async_copy(src, tail_o.at[dst], rowsem)

                    @pl.when(mi & (tbl == 2))
                    def _():
                        pltpu.async_copy(src, neg_o.at[dst], rowsem)
                    cur = cur + mint[i]
                iss2 = cur

                def drain_some(st4):
                    iss4, drn4 = st4

                    def dr(i2, d):
                        pltpu.make_async_copy(dummy, ring.at[0], rowsem).wait()
                        return d + 1
                    drn4 = lax.fori_loop(0, 16, dr, drn4)
                    return (iss4, drn4)
                return lax.cond(iss2 - drn2 >= _RING - 32, drain_some,
                                lambda s: s, (iss2, drn2))
            return lax.cond(n > 0, have, lambda s: s, st)
        return lax.fori_loop(0, (seg_cnt + 15) >> 4, kchunk, state)

    bufs = (chunk0, chunk1, chunk2, chunk3)
    sems = (csem0, csem1, csem2, csem3)

    def want(c):
        return (c < ncols) & (colsm[jnp.minimum(c, 255)] > 0)

    def fire(c, q):
        @pl.when(want(c))
        def _():
            pltpu.async_copy(etT.at[:, pl.ds((t0 + c) * 128, 128)],
                             bufs[q], sems[q])

    for q in range(3):
        fire(q, q)

    def quadbody(i, st):
        for q in range(4):
            c = 4 * i + q
            fire(c + 3, (q + 3) % 4)

            def do(st2, c=c, q=q):
                pltpu.make_async_copy(etT.at[:, pl.ds(0, 128)], bufs[q],
                                      sems[q]).wait()
                cm = jnp.minimum(c, 255)
                return process_col(colstart[cm], colsm[cm], bufs[q], st2)
            st = lax.cond(want(c), do, lambda s2: s2, st)
        return st

    issued, drained = lax.fori_loop(0, 62, quadbody,
                                    (jnp.int32(0), jnp.int32(0)))

    def drfin(i, d):
        pltpu.make_async_copy(dummy, ring.at[0], rowsem).wait()
        return d + 1
    lax.fori_loop(0, issued - drained, drfin, drained)

    for g in small:
        g.wait()
    dD.wait()
    stores = [
        st_rel, st_ctx,
        pltpu.async_copy(crow, curv_o.at[sl], ssem),
        pltpu.async_copy(lineA, diag_o.at[sl], ssem),
        pltpu.async_copy(hbrow, hb_o.at[sl], ssem),
        pltpu.async_copy(tbrow, tb_o.at[sl], ssem),
        pltpu.async_copy(ntbrow, ntb_o.at[sl], ssem),
    ]
    for s in stores:
        s.wait()


def kernel(entity_emb, relation_emb, relation_diag, curvature, context,
           head_bias, tail_bias, head, tail, rel, neg):
    scale = jnp.array([0.125], dtype=jnp.float32)
    (head_e, tail_e, rel_e, neg_e, curv, rel_diag, ctx,
     h_bias, t_bias, neg_t_bias) = _gather_all(
        entity_emb.T,
        relation_emb.reshape(N_RELATION // 2, 2 * HIDDEN),
        relation_diag,
        curvature.reshape(N_RELATION),
        context.reshape(N_RELATION // 2, 2 * HIDDEN),
        head_bias.reshape(N_ENTITY), tail_bias.reshape(N_ENTITY),
        head.astype(jnp.int32), tail.astype(jnp.int32),
        rel.astype(jnp.int32), neg.astype(jnp.int32))
    return (scale, head_e.reshape(BATCH, HIDDEN), tail_e.reshape(BATCH, HIDDEN),
            rel_e, neg_e.reshape(BATCH, HIDDEN),
            curv.reshape(BATCH, 1), rel_diag, ctx,
            h_bias.reshape(BATCH, 1), t_bias.reshape(BATCH, 1),
            neg_t_bias.reshape(BATCH, 1))


# speedup vs baseline: 2.0093x; 1.4937x over previous
import functools

import jax
import jax.numpy as jnp
from jax import lax
from jax.experimental import pallas as pl
from jax.experimental.pallas import tpu as pltpu
from jax.experimental.pallas import tpu_sc as plsc

N_ENTITY = 1000000
N_RELATION = 1000
HIDDEN = 64
BATCH = 4096

_NC, _NS = 2, 16
_NW = _NC * _NS
_BW = BATCH // _NW
_NTC = (N_ENTITY + 127) // 128
_KCAP = 3 * BATCH
_RING = 64

_mesh = plsc.VectorSubcoreMesh(core_axis_name="c", subcore_axis_name="s")


@functools.partial(
    pl.kernel,
    mesh=_mesh,
    compiler_params=pltpu.CompilerParams(use_tc_tiling_on_sc=True,
                                         needs_layout_passes=False),
    out_type=(
        jax.ShapeDtypeStruct((BATCH * HIDDEN,), jnp.float32),
        jax.ShapeDtypeStruct((BATCH * HIDDEN,), jnp.float32),
        jax.ShapeDtypeStruct((BATCH, HIDDEN), jnp.float32),
        jax.ShapeDtypeStruct((BATCH * HIDDEN,), jnp.float32),
        jax.ShapeDtypeStruct((BATCH,), jnp.float32),
        jax.ShapeDtypeStruct((BATCH, 2 * HIDDEN), jnp.float32),
        jax.ShapeDtypeStruct((BATCH, HIDDEN), jnp.float32),
        jax.ShapeDtypeStruct((BATCH,), jnp.float32),
        jax.ShapeDtypeStruct((BATCH,), jnp.float32),
        jax.ShapeDtypeStruct((BATCH,), jnp.float32),
    ),
    scratch_types=(
        pltpu.VMEM((_BW,), jnp.int32),
        pltpu.VMEM((_BW,), jnp.int32),
        pltpu.VMEM((_BW,), jnp.int32),
        pltpu.VMEM((_BW,), jnp.int32),
        pltpu.VMEM((_BW,), jnp.int32),
        pltpu.VMEM((BATCH,), jnp.int32),
        pltpu.VMEM((BATCH,), jnp.int32),
        pltpu.VMEM((BATCH,), jnp.int32),
        pltpu.VMEM((_KCAP,), jnp.int32),
        pltpu.VMEM((_KCAP,), jnp.int32),
        pltpu.SMEM((256,), jnp.int32),
        pltpu.SMEM((256,), jnp.int32),
        pltpu.SMEM((256,), jnp.int32),
        pltpu.VMEM((64, 128), jnp.float32),
        pltpu.VMEM((64, 128), jnp.float32),
        pltpu.VMEM((_RING, HIDDEN), jnp.float32),
        pltpu.VMEM((_BW, 128), jnp.float32),
        pltpu.VMEM((_BW, HIDDEN), jnp.float32),
        pltpu.VMEM((_BW, HIDDEN), jnp.float32),
        pltpu.VMEM((_BW,), jnp.float32),
        pltpu.VMEM((_BW,), jnp.float32),
        pltpu.VMEM((_BW,), jnp.float32),
        pltpu.VMEM((_BW,), jnp.float32),
        pltpu.SemaphoreType.DMA,
        pltpu.SemaphoreType.DMA,
        pltpu.SemaphoreType.DMA,
        pltpu.SemaphoreType.DMA,
        pltpu.SemaphoreType.DMA,
        pltpu.SemaphoreType.DMA,
    ),
)
def _gather_all(etT, rel2, diag, curv1, ctx2, hb1, tb1, head, tail, rel, neg,
                head_o, tail_o, rel_o, neg_o, curv_o, diag_o, ctx_o,
                hb_o, tb_o, ntb_o,
                ridx, rp, hidx, tidx, nidx, hfull, tfull, nfull,
                hits, hits2, colsm, colstart, coloff, chunk0, chunk1, ring,
                lineA, rrow, xrow, crow, hbrow, tbrow, ntbrow,
                gsem, semA, csem0, csem1, rowsem, ssem):
    wid = lax.axis_index("s") * _NC + lax.axis_index("c")
    base = wid * _BW
    sl = pl.ds(base, _BW)
    iota16 = lax.iota(jnp.int32, 16)

    pltpu.sync_copy(rel.at[sl], ridx)
    pltpu.sync_copy(head.at[sl], hidx)
    pltpu.sync_copy(tail.at[sl], tidx)
    pltpu.sync_copy(neg.at[sl], nidx)

    def mkpairs(i, _):
        s16 = pl.ds(i * 16, 16)
        rp[s16] = ridx[s16] >> 1
        return 0
    lax.fori_loop(0, _BW // 16, mkpairs, 0, unroll=True)

    small = [
        pltpu.async_copy(curv1.at[ridx], crow, gsem),
        pltpu.async_copy(hb1.at[hidx], hbrow, gsem),
        pltpu.async_copy(tb1.at[tidx], tbrow, gsem),
        pltpu.async_copy(tb1.at[nidx], ntbrow, gsem),
    ]
    dA = pltpu.async_copy(rel2.at[rp], lineA, semA)

    def sel_table(line, idxv, row):
        for k in range(_BW // 16):
            hvec = iota16 + (16 * k)
            colb = (idxv[pl.ds(16 * k, 16)] & 1) * HIDDEN

            def cbody(c, _):
                v = plsc.load_gather(line, [hvec, colb + c])
                cvec = jnp.full((16,), c, dtype=jnp.int32)
                plsc.store_scatter(row, [hvec, cvec], v)
                return 0
            lax.fori_loop(0, HIDDEN, cbody, 0)

    pltpu.sync_copy(head, hfull)
    pltpu.sync_copy(tail, tfull)
    pltpu.sync_copy(neg, nfull)

    t0 = (wid * _NTC) // _NW
    t1 = ((wid + 1) * _NTC) // _NW

    def build(tblref, tblid, kcnt0):
        def chunk(j, kcnt):
            r = tblref[pl.ds(j * 16, 16)]
            tc = r >> 7
            m = (tc >= t0) & (tc < t1)
            n = lax.reduce_max(plsc.all_reduce_population_count(m), (0,))

            def have():
                pos = kcnt + plsc.cumsum(m.astype(jnp.int32)) - 1
                ea = ((tc - t0) << 7) | (r & 127)
                eb = (16 * j + iota16) | (tblid << 12)
                plsc.store_scatter(hits, [pos], ea | (eb << 15), mask=m)
            pl.when(n > 0)(have)
            return kcnt + n
        return lax.fori_loop(0, BATCH // 16, chunk, kcnt0)

    kcnt = build(hfull, 0, jnp.int32(0))
    kcnt = build(tfull, 1, kcnt)
    kcnt = build(nfull, 2, kcnt)
    kchunks = (kcnt + 15) >> 4

    ncols = t1 - t0
    dummy = hb1.at[pl.ds(0, HIDDEN)]
    zeros16 = jnp.zeros((16,), jnp.int32)

    def zerocol(i, _):
        colsm[i] = 0
        return 0
    lax.fori_loop(0, 256, zerocol, 0)

    def count_chunk(j, _):
        a = hits[pl.ds(j * 16, 16)]
        valid = ((j * 16 + iota16) < kcnt).astype(jnp.int32)
        trel = (a & 0x7FFF) >> 7
        for i in range(16):
            colsm[trel[i]] = colsm[trel[i]] + valid[i]
        return 0
    lax.fori_loop(0, kchunks, count_chunk, 0)

    def prefix(i, run):
        colstart[i] = run
        coloff[i] = run
        return run + colsm[i]
    lax.fori_loop(0, 256, prefix, jnp.int32(0))

    def place_chunk(j, _):
        a = hits[pl.ds(j * 16, 16)]
        validb = (j * 16 + iota16) < kcnt
        valid = validb.astype(jnp.int32)
        trel = (a & 0x7FFF) >> 7
        pos = zeros16
        for i in range(16):
            p = coloff[trel[i]]
            coloff[trel[i]] = p + valid[i]
            pos = jnp.where(iota16 == i, p, pos)
        plsc.store_scatter(hits2, [pos], a, mask=validb)
        return 0
    lax.fori_loop(0, kchunks, place_chunk, 0)

    dA.wait()
    sel_table(lineA, ridx, rrow)
    dX = pltpu.async_copy(ctx2.at[rp], lineA, semA)
    st_rel = pltpu.async_copy(rrow, rel_o.at[sl], ssem)
    dX.wait()
    sel_table(lineA, ridx, xrow)
    dD = pltpu.async_copy(diag.at[ridx], lineA, semA)
    st_ctx = pltpu.async_copy(xrow, ctx_o.at[sl], ssem)

    def process_col(seg_base, seg_cnt, chunk, state):
        def kchunk(j, st):
            a = hits2[pl.ds(seg_base + j * 16, 16)]
            m = (j * 16 + iota16) < seg_cnt
            n = jnp.minimum(seg_cnt - j * 16, 16)

            def have(st2):
                iss2, drn2 = st2
                cur = iss2
                for i in range(16):
                    mi = (j * 16 + i) < seg_cnt
                    e = a[i]
                    b = (e >> 15) & 4095
                    tbl = e >> 27
                    slot = cur & (_RING - 1)

                    @pl.when(mi)
                    def _():
                        lbc = jnp.full((16,), e & 127, dtype=jnp.int32)
                        for q in range(4):
                            v = plsc.load_gather(chunk, [iota16 + 16 * q, lbc])
                            ring[slot, pl.ds(16 * q, 16)] = v
                    src = ring.at[slot]
                    dst = pl.ds(b * HIDDEN, HIDDEN)

                    @pl.when(mi & (tbl == 0))
                    def _():
                        pltpu.async_copy(src, head_o.at[dst], rowsem)

                    @pl.when(mi & (tbl == 1))
                    def _():
                        pltpu.async_copy(src, tail_o.at[dst], rowsem)

                    @pl.when(mi & (tbl == 2))
                    def _():
                        pltpu.async_copy(src, neg_o.at[dst], rowsem)
                    cur = cur + mi.astype(jnp.int32)
                iss2 = cur

                def drain_some(st4):
                    iss4, drn4 = st4

                    def dr(i2, d):
                        pltpu.make_async_copy(dummy, ring.at[0], rowsem).wait()
                        return d + 1
                    drn4 = lax.fori_loop(0, 16, dr, drn4)
                    return (iss4, drn4)
                return lax.cond(iss2 - drn2 >= _RING - 32, drain_some,
                                lambda s: s, (iss2, drn2))
            return lax.cond(n > 0, have, lambda s: s, st)
        return lax.fori_loop(0, (seg_cnt + 15) >> 4, kchunk, state)

    @pl.when(0 < ncols)
    def _():
        pltpu.async_copy(etT.at[:, pl.ds(t0 * 128, 128)], chunk0, csem0)

    def pairbody(i, st):
        cA = 2 * i
        cB = cA + 1

        @pl.when(cB < ncols)
        def _():
            pltpu.async_copy(etT.at[:, pl.ds((t0 + cB) * 128, 128)],
                             chunk1, csem1)

        def doA(st2):
            pltpu.make_async_copy(etT.at[:, pl.ds(0, 128)], chunk0,
                                  csem0).wait()
            st2 = process_col(colstart[cA], colsm[cA], chunk0, st2)

            @pl.when(cB + 1 < ncols)
            def _():
                pltpu.async_copy(etT.at[:, pl.ds((t0 + cB + 1) * 128, 128)],
                                 chunk0, csem0)
            return st2
        st = lax.cond(cA < ncols, doA, lambda s2: s2, st)

        def doB(st2):
            pltpu.make_async_copy(etT.at[:, pl.ds(0, 128)], chunk1,
                                  csem1).wait()
            return process_col(colstart[cB], colsm[cB], chunk1, st2)
        return lax.cond(cB < ncols, doB, lambda s2: s2, st)

    issued, drained = lax.fori_loop(0, (_NTC // _NW) // 2 + 2, pairbody,
                                    (jnp.int32(0), jnp.int32(0)))

    def drfin(i, d):
        pltpu.make_async_copy(dummy, ring.at[0], rowsem).wait()
        return d + 1
    lax.fori_loop(0, issued - drained, drfin, drained)

    for g in small:
        g.wait()
    dD.wait()
    stores = [
        st_rel, st_ctx,
        pltpu.async_copy(crow, curv_o.at[sl], ssem),
        pltpu.async_copy(lineA, diag_o.at[sl], ssem),
        pltpu.async_copy(hbrow, hb_o.at[sl], ssem),
        pltpu.async_copy(tbrow, tb_o.at[sl], ssem),
        pltpu.async_copy(ntbrow, ntb_o.at[sl], ssem),
    ]
    for s in stores:
        s.wait()


def kernel(entity_emb, relation_emb, relation_diag, curvature, context,
           head_bias, tail_bias, head, tail, rel, neg):
    scale = jnp.array([0.125], dtype=jnp.float32)
    (head_e, tail_e, rel_e, neg_e, curv, rel_diag, ctx,
     h_bias, t_bias, neg_t_bias) = _gather_all(
        entity_emb.T,
        relation_emb.reshape(N_RELATION // 2, 2 * HIDDEN),
        relation_diag,
        curvature.reshape(N_RELATION),
        context.reshape(N_RELATION // 2, 2 * HIDDEN),
        head_bias.reshape(N_ENTITY), tail_bias.reshape(N_ENTITY),
        head.astype(jnp.int32), tail.astype(jnp.int32),
        rel.astype(jnp.int32), neg.astype(jnp.int32))
    return (scale, head_e.reshape(BATCH, HIDDEN), tail_e.reshape(BATCH, HIDDEN),
            rel_e, neg_e.reshape(BATCH, HIDDEN),
            curv.reshape(BATCH, 1), rel_diag, ctx,
            h_bias.reshape(BATCH, 1), t_bias.reshape(BATCH, 1),
            neg_t_bias.reshape(BATCH, 1))
